# baseline (speedup 1.0000x reference)
import jax
import jax.numpy as jnp
from jax import lax
from jax.experimental import pallas as pl
from jax.experimental.pallas import tpu as pltpu

B = 2048
IN_DIM = 2048
N_GENES = 512
N_DONORS = 8
M_BLK = 128
N_BLOCKS = B // M_BLK + N_DONORS - 1
PAD_ROWS = N_BLOCKS * M_BLK
GRP = 16
GLANES = B // GRP


def _routing_body(labels_ref, ppos_ref, gid_ref):
    labels = labels_ref[...]
    lab128 = jnp.concatenate([labels] * N_DONORS, axis=0)
    r_iota = lax.broadcasted_iota(jnp.int32, (N_DONORS * GRP, GLANES), 0)
    d_of_row = r_iota // GRP
    oh = (lab128 == d_of_row).astype(jnp.bfloat16)

    j_a = lax.broadcasted_iota(jnp.int32, (GLANES, GLANES), 0)
    j_b = lax.broadcasted_iota(jnp.int32, (GLANES, GLANES), 1)
    upper_incl = (j_a <= j_b).astype(jnp.bfloat16)
    intra = lax.dot_general(oh, upper_incl, (((1,), (0,)), ((), ())),
                            preferred_element_type=jnp.float32)
    total = intra[:, GLANES - 1:GLANES]

    s_a = lax.broadcasted_iota(jnp.int32, (N_DONORS * GRP, N_DONORS * GRP), 0)
    s_b = lax.broadcasted_iota(jnp.int32, (N_DONORS * GRP, N_DONORS * GRP), 1)
    same_band = (s_a // GRP) == (s_b // GRP)
    strict = jnp.logical_and(same_band, s_b < s_a).astype(jnp.bfloat16)
    grpoff = lax.dot_general(strict, total.astype(jnp.bfloat16),
                             (((1,), (0,)), ((), ())),
                             preferred_element_type=jnp.float32)
    csum = intra + grpoff

    e_d = lax.broadcasted_iota(jnp.int32, (N_DONORS, N_DONORS * GRP), 0)
    e_r = lax.broadcasted_iota(jnp.int32, (N_DONORS, N_DONORS * GRP), 1)
    band_sel = (e_d == e_r // GRP).astype(jnp.bfloat16)
    counts = lax.dot_general(band_sel, total.astype(jnp.bfloat16),
                             (((1,), (0,)), ((), ())),
                             preferred_element_type=jnp.float32)

    nblk = jnp.floor((counts + (M_BLK - 1)) * (1.0 / M_BLK))
    t_a = lax.broadcasted_iota(jnp.int32, (N_DONORS, N_DONORS), 0)
    t_b = lax.broadcasted_iota(jnp.int32, (N_DONORS, N_DONORS), 1)
    s8 = (t_b < t_a).astype(jnp.bfloat16)
    pblk_off = lax.dot_general(s8, nblk.astype(jnp.bfloat16),
                               (((1,), (0,)), ((), ())),
                               preferred_element_type=jnp.float32)

    f_r = lax.broadcasted_iota(jnp.int32, (N_DONORS * GRP, N_DONORS), 0)
    f_d = lax.broadcasted_iota(jnp.int32, (N_DONORS * GRP, N_DONORS), 1)
    tile_sel = (f_r // GRP == f_d).astype(jnp.bfloat16)
    poff128 = lax.dot_general(tile_sel, pblk_off.astype(jnp.bfloat16),
                              (((1,), (0,)), ((), ())),
                              preferred_element_type=jnp.float32) * float(M_BLK)

    pre = oh.astype(jnp.float32) * (csum - 1.0 + poff128)
    ppos = jnp.sum(pre.reshape(N_DONORS, GRP, GLANES), axis=0)
    ppos_ref[...] = ppos.astype(jnp.int32)

    blk_iota = lax.broadcasted_iota(jnp.int32, (N_DONORS, 128), 1)
    cmp = (pblk_off.astype(jnp.int32) <= blk_iota).astype(jnp.int32)
    gid_ref[...] = jnp.sum(cmp, axis=0, keepdims=True) - 1


def _routing(labels):
    return pl.pallas_call(
        _routing_body,
        in_specs=[pl.BlockSpec((GRP, GLANES), lambda: (0, 0))],
        out_specs=[
            pl.BlockSpec((GRP, GLANES), lambda: (0, 0)),
            pl.BlockSpec((1, 128), lambda: (0, 0)),
        ],
        out_shape=[
            jax.ShapeDtypeStruct((GRP, GLANES), jnp.int32),
            jax.ShapeDtypeStruct((1, 128), jnp.int32),
        ],
    )(labels.reshape(GRP, GLANES))


def _dispatch_body(gid_ref, ppos_ref, x_hbm, w_ref, b_ref, out_ref,
                   xs_ref, y_ref, sem_x, sem_y):
    i = pl.program_id(0)

    @pl.when(i == 0)
    def _scatter_x():
        def issue(t, c):
            g = t // GLANES
            j = t - g * GLANES
            dst = ppos_ref[g, j]
            pltpu.make_async_copy(
                x_hbm.at[pl.ds(t, 1), :], xs_ref.at[pl.ds(dst, 1), :], sem_x
            ).start()
            return c

        lax.fori_loop(0, B, issue, 0, unroll=8)

        def drain(k, c):
            pltpu.make_async_copy(
                x_hbm.at[pl.ds(0, M_BLK), :],
                xs_ref.at[pl.ds(0, M_BLK), :], sem_x,
            ).wait()
            return c

        lax.fori_loop(0, B // M_BLK, drain, 0, unroll=1)

    b_row = b_ref[pl.ds(gid_ref[0, i], 1), :]
    y_ref[pl.ds(i * M_BLK, M_BLK), :] = lax.dot_general(
        xs_ref[pl.ds(i * M_BLK, M_BLK), :], w_ref[0],
        dimension_numbers=(((1,), (1,)), ((), ())),
        preferred_element_type=jnp.float32,
    ) + b_row

    @pl.when(i == N_BLOCKS - 1)
    def _unpermute_y():
        def issue(t, c):
            g = t // GLANES
            j = t - g * GLANES
            src = ppos_ref[g, j]
            pltpu.make_async_copy(
                y_ref.at[pl.ds(src, 1), :], out_ref.at[pl.ds(t, 1), :], sem_y
            ).start()
            return c

        lax.fori_loop(0, B, issue, 0, unroll=8)

        def drain(k, c):
            pltpu.make_async_copy(
                y_ref.at[pl.ds(0, M_BLK), :],
                out_ref.at[pl.ds(0, M_BLK), :], sem_y,
            ).wait()
            return c

        lax.fori_loop(0, B // M_BLK, drain, 0, unroll=1)


def _dispatch(x, W, b, ppos2d, gid2d):
    grid_spec = pltpu.PrefetchScalarGridSpec(
        num_scalar_prefetch=1,
        grid=(N_BLOCKS,),
        in_specs=[
            pl.BlockSpec(memory_space=pltpu.SMEM),
            pl.BlockSpec(memory_space=pl.ANY),
            pl.BlockSpec((1, N_GENES, IN_DIM), lambda i, g: (g[0, i], 0, 0)),
            pl.BlockSpec((N_DONORS, N_GENES), lambda i, g: (0, 0)),
        ],
        out_specs=pl.BlockSpec((B, N_GENES), lambda i, g: (0, 0)),
        scratch_shapes=[
            pltpu.VMEM((PAD_ROWS, IN_DIM), jnp.float32),
            pltpu.VMEM((PAD_ROWS, N_GENES), jnp.float32),
            pltpu.SemaphoreType.DMA,
            pltpu.SemaphoreType.DMA,
        ],
    )
    return pl.pallas_call(
        _dispatch_body,
        grid_spec=grid_spec,
        out_shape=jax.ShapeDtypeStruct((B, N_GENES), jnp.float32),
    )(gid2d, ppos2d, x, W, b)


def kernel(x, donor_labels, W, b):
    ppos2d, gid2d = _routing(donor_labels)
    return _dispatch(x, W, b, ppos2d, gid2d)


# speedup vs baseline: 1.8703x; 1.8703x over previous
import jax
import jax.numpy as jnp
from jax import lax
from jax.experimental import pallas as pl

B = 2048
IN_DIM = 2048
N_GENES = 512
N_DONORS = 8
TOK_BLK = 256


def _dense_body(labels_ref, x_ref, w_ref, b_ref, out_ref):
    labels = labels_ref[...]
    acc = jnp.zeros(out_ref.shape, dtype=jnp.float32)
    for d in range(N_DONORS):
        pred = lax.dot_general(
            x_ref[...], w_ref[d],
            dimension_numbers=(((1,), (1,)), ((), ())),
            preferred_element_type=jnp.float32,
        ) + b_ref[d][None, :]
        acc = jnp.where(labels == d, pred, acc)
    out_ref[...] = acc


def kernel(x, donor_labels, W, b):
    n_blocks = B // TOK_BLK
    labels2 = donor_labels.reshape(B, 1)
    return pl.pallas_call(
        _dense_body,
        grid=(n_blocks,),
        in_specs=[
            pl.BlockSpec((TOK_BLK, 1), lambda i: (i, 0)),
            pl.BlockSpec((TOK_BLK, IN_DIM), lambda i: (i, 0)),
            pl.BlockSpec((N_DONORS, N_GENES, IN_DIM), lambda i: (0, 0, 0)),
            pl.BlockSpec((N_DONORS, N_GENES), lambda i: (0, 0)),
        ],
        out_specs=pl.BlockSpec((TOK_BLK, N_GENES), lambda i: (i, 0)),
        out_shape=jax.ShapeDtypeStruct((B, N_GENES), jnp.float32),
    )(labels2, x, W, b)
